# baseline (speedup 1.0000x reference)
import functools

import numpy as np

import jax
import jax.numpy as jnp
from jax import lax
from jax.experimental import pallas as pl
from jax.experimental.pallas import tpu as pltpu
from jax.experimental.pallas import tpu_sc as plsc

N = 10000
D = 128
H = 32
E = 320000

LANES = 128
GROUP = 16
NC = 2
NS = 16
NW = NC * NS
B0 = 48
B1 = 112
BPW = max(B0, B1)
NB = NS * (B0 + B1)
NBP = NB + (BPW - min(B0, B1))
EP = NBP * LANES
NPAD = 10112
RPT = NPAD // NS
DUMMY = 10104
CW = 16

ROWS_BLK = 2000

_PAD_SRC = np.zeros((EP - E,), np.int32)
_PAD_DST = (N + np.arange(EP - E, dtype=np.int32) % (NPAD - N)).astype(np.int32)


def _sc_mesh():
    return plsc.VectorSubcoreMesh(core_axis_name="c", subcore_axis_name="s")


@functools.partial(
    pl.kernel,
    mesh=_sc_mesh(),
    compiler_params=pltpu.CompilerParams(use_tc_tiling_on_sc=False),
    out_type=[
        jax.ShapeDtypeStruct((NC, NPAD, H), jnp.float32),
        jax.ShapeDtypeStruct((NC, NPAD, CW), jnp.float32),
    ],
    scratch_types=[
        pltpu.VMEM((BPW, LANES), jnp.int32),
        pltpu.VMEM((BPW, LANES), jnp.int32),
        pltpu.VMEM((GROUP, LANES, H), jnp.float32),
        pltpu.VMEM((LANES, CW), jnp.float32),
        pltpu.VMEM_SHARED((NPAD, H), jnp.float32),
        pltpu.VMEM_SHARED((NPAD, CW), jnp.float32),
        pltpu.SemaphoreType.DMA((GROUP,)),
    ],
)
def _sc_agg_cnt(y_hbm, src_hbm, dst_hbm, zacc_hbm, zcnt_hbm, ones_hbm,
                out_acc, out_cnt, src_v, dst_v, bufs, ones_v, acc_s, cnt_s,
                sems):
    c = lax.axis_index("c")
    s = lax.axis_index("s")
    base_b = jnp.where(c == 0, s * B0, NS * B0 + s * B1)
    nb_w = jnp.where(c == 0, B0, B1)
    r0 = s * RPT
    pltpu.sync_copy(zacc_hbm.at[pl.ds(r0, RPT)], acc_s.at[pl.ds(r0, RPT)])
    pltpu.sync_copy(zcnt_hbm.at[pl.ds(r0, RPT)], cnt_s.at[pl.ds(r0, RPT)])
    pltpu.sync_copy(ones_hbm, ones_v)
    pltpu.sync_copy(src_hbm.at[pl.ds(base_b, BPW)], src_v)
    pltpu.sync_copy(dst_hbm.at[pl.ds(base_b, BPW)], dst_v)
    plsc.subcore_barrier()

    ng = nb_w // GROUP
    for b in range(GROUP):
        pltpu.async_copy(y_hbm.at[src_v.at[b]], bufs.at[b], sems.at[b])

    def group(g, carry):
        base = g * GROUP
        for b in range(GROUP):
            pltpu.make_async_copy(y_hbm.at[src_v.at[base + b]], bufs.at[b],
                                  sems.at[b]).wait()
            pltpu.sync_copy(bufs.at[b], acc_s.at[dst_v.at[base + b]],
                            add=True)
            pltpu.sync_copy(ones_v, cnt_s.at[dst_v.at[base + b]], add=True)

            @pl.when(g + 1 < ng)
            def _():
                pltpu.async_copy(y_hbm.at[src_v.at[base + GROUP + b]],
                                 bufs.at[b], sems.at[b])
        return carry

    lax.fori_loop(0, ng, group, 0)
    plsc.subcore_barrier()
    pltpu.sync_copy(acc_s.at[pl.ds(r0, RPT)], out_acc.at[c, pl.ds(r0, RPT)])
    pltpu.sync_copy(cnt_s.at[pl.ds(r0, RPT)], out_cnt.at[c, pl.ds(r0, RPT)])


@functools.partial(
    pl.kernel,
    mesh=_sc_mesh(),
    compiler_params=pltpu.CompilerParams(use_tc_tiling_on_sc=False),
    out_type=jax.ShapeDtypeStruct((NC, NPAD, H), jnp.float32),
    scratch_types=[
        pltpu.VMEM((BPW, LANES), jnp.int32),
        pltpu.VMEM((BPW, LANES), jnp.int32),
        pltpu.VMEM((GROUP, LANES, H), jnp.float32),
        pltpu.VMEM_SHARED((NPAD, H), jnp.float32),
        pltpu.SemaphoreType.DMA((GROUP,)),
    ],
)
def _sc_agg(y_hbm, src_hbm, dst_hbm, zacc_hbm, out_acc, src_v, dst_v, bufs,
            acc_s, sems):
    c = lax.axis_index("c")
    s = lax.axis_index("s")
    base_b = jnp.where(c == 0, s * B0, NS * B0 + s * B1)
    nb_w = jnp.where(c == 0, B0, B1)
    r0 = s * RPT
    pltpu.sync_copy(zacc_hbm.at[pl.ds(r0, RPT)], acc_s.at[pl.ds(r0, RPT)])
    pltpu.sync_copy(src_hbm.at[pl.ds(base_b, BPW)], src_v)
    pltpu.sync_copy(dst_hbm.at[pl.ds(base_b, BPW)], dst_v)
    plsc.subcore_barrier()

    ng = nb_w // GROUP
    for b in range(GROUP):
        pltpu.async_copy(y_hbm.at[src_v.at[b]], bufs.at[b], sems.at[b])

    def group(g, carry):
        base = g * GROUP
        for b in range(GROUP):
            pltpu.make_async_copy(y_hbm.at[src_v.at[base + b]], bufs.at[b],
                                  sems.at[b]).wait()
            pltpu.sync_copy(bufs.at[b], acc_s.at[dst_v.at[base + b]],
                            add=True)

            @pl.when(g + 1 < ng)
            def _():
                pltpu.async_copy(y_hbm.at[src_v.at[base + GROUP + b]],
                                 bufs.at[b], sems.at[b])
        return carry

    lax.fori_loop(0, ng, group, 0)
    plsc.subcore_barrier()
    pltpu.sync_copy(acc_s.at[pl.ds(r0, RPT)], out_acc.at[c, pl.ds(r0, RPT)])


def _proj_body(x_ref, w_ref, oy_ref, or_ref):
    t = jnp.dot(x_ref[...], w_ref[...], preferred_element_type=jnp.float32)
    oy_ref[...] = t[:, :H]
    or_ref[...] = t[:, H:]


def _proj(x, wcat):
    return pl.pallas_call(
        _proj_body,
        grid=(N // ROWS_BLK,),
        in_specs=[
            pl.BlockSpec((ROWS_BLK, D), lambda i: (i, 0)),
            pl.BlockSpec((D, 2 * H), lambda i: (0, 0)),
        ],
        out_specs=[
            pl.BlockSpec((ROWS_BLK, H), lambda i: (i, 0)),
            pl.BlockSpec((ROWS_BLK, H), lambda i: (i, 0)),
        ],
        out_shape=[
            jax.ShapeDtypeStruct((N, H), jnp.float32),
            jax.ShapeDtypeStruct((N, H), jnp.float32),
        ],
    )(x, wcat)


def _mid_body(p_ref, c_ref, r_ref, b_ref, w_ref, oy_ref, or_ref, oc_ref):
    p = p_ref[...]
    cc = c_ref[...]
    cnt = jnp.maximum(cc[0] + cc[1], 1.0)
    agg = (p[0] + p[1]) / cnt[:, 0:1]
    h = jnp.maximum(agg + b_ref[...] + r_ref[...], 0.0)
    t = jnp.dot(h, w_ref[...], preferred_element_type=jnp.float32)
    oy_ref[...] = t[:, :H]
    or_ref[...] = t[:, H:]
    oc_ref[...] = cnt


def _mid(psum, pcnt, r1, b1, wcat2):
    blk_h = lambda i: (i, 0)
    return pl.pallas_call(
        _mid_body,
        grid=(N // ROWS_BLK,),
        in_specs=[
            pl.BlockSpec((NC, ROWS_BLK, H), lambda i: (0, i, 0)),
            pl.BlockSpec((NC, ROWS_BLK, CW), lambda i: (0, i, 0)),
            pl.BlockSpec((ROWS_BLK, H), blk_h),
            pl.BlockSpec((1, H), lambda i: (0, 0)),
            pl.BlockSpec((H, 2 * H), lambda i: (0, 0)),
        ],
        out_specs=[
            pl.BlockSpec((ROWS_BLK, H), blk_h),
            pl.BlockSpec((ROWS_BLK, H), blk_h),
            pl.BlockSpec((ROWS_BLK, CW), blk_h),
        ],
        out_shape=[
            jax.ShapeDtypeStruct((N, H), jnp.float32),
            jax.ShapeDtypeStruct((N, H), jnp.float32),
            jax.ShapeDtypeStruct((N, CW), jnp.float32),
        ],
    )(psum, pcnt, r1, b1, wcat2)


def _fin_body(p_ref, c_ref, r_ref, b_ref, o_ref):
    p = p_ref[...]
    o_ref[...] = ((p[0] + p[1]) / c_ref[:, 0:1] + b_ref[...] + r_ref[...])


def _fin(psum, cnt, r2, b2):
    blk_h = lambda i: (i, 0)
    return pl.pallas_call(
        _fin_body,
        grid=(N // ROWS_BLK,),
        in_specs=[
            pl.BlockSpec((NC, ROWS_BLK, H), lambda i: (0, i, 0)),
            pl.BlockSpec((ROWS_BLK, CW), blk_h),
            pl.BlockSpec((ROWS_BLK, H), blk_h),
            pl.BlockSpec((1, H), lambda i: (0, 0)),
        ],
        out_specs=pl.BlockSpec((ROWS_BLK, H), blk_h),
        out_shape=jax.ShapeDtypeStruct((N, H), jnp.float32),
    )(psum, cnt, r2, b2)


def kernel(x, edge_index, W1_l, b1, W1_r, W2_l, b2, W2_r):
    src = edge_index[0]
    dst = edge_index[1]
    src_p = jnp.concatenate([src, jnp.asarray(_PAD_SRC)]).reshape(NBP, LANES)
    dst_p = jnp.concatenate([dst, jnp.asarray(_PAD_DST)]).reshape(NBP, LANES)
    zacc = jnp.zeros((NPAD, H), jnp.float32)
    zcnt = jnp.zeros((NPAD, CW), jnp.float32)
    ones = jnp.ones((LANES, CW), jnp.float32)

    y1, r1 = _proj(x, jnp.concatenate([W1_l, W1_r], axis=1))
    psum1, pcnt = _sc_agg_cnt(y1, src_p, dst_p, zacc, zcnt, ones)
    y2, r2, cnt = _mid(psum1, pcnt, r1, b1.reshape(1, H),
                       jnp.concatenate([W2_l, W2_r], axis=1))

    psum2 = _sc_agg(y2, src_p, dst_p, zacc)
    return _fin(psum2, cnt, r2, b2.reshape(1, H))


# speedup vs baseline: 1.0460x; 1.0460x over previous
import functools

import numpy as np

import jax
import jax.numpy as jnp
from jax import lax
from jax.experimental import pallas as pl
from jax.experimental.pallas import tpu as pltpu
from jax.experimental.pallas import tpu_sc as plsc

N = 10000
D = 128
H = 32
E = 320000

LANES = 128
GROUP = 16
NC = 2
NS = 16
NW = NC * NS
B0 = 80
B1 = 80
BPW = max(B0, B1)
NB = NS * (B0 + B1)
NBP = NB + (BPW - min(B0, B1))
EP = NBP * LANES
NPAD = 10112
RPT = NPAD // NS
DUMMY = 10104
CW = 16

ROWS_BLK = 2000

_PAD_SRC = np.zeros((EP - E,), np.int32)
_PAD_DST = (N + np.arange(EP - E, dtype=np.int32) % (NPAD - N)).astype(np.int32)


def _sc_mesh():
    return plsc.VectorSubcoreMesh(core_axis_name="c", subcore_axis_name="s")


@functools.partial(
    pl.kernel,
    mesh=_sc_mesh(),
    compiler_params=pltpu.CompilerParams(use_tc_tiling_on_sc=False),
    out_type=[
        jax.ShapeDtypeStruct((NC, NPAD, H), jnp.float32),
        jax.ShapeDtypeStruct((NC, NPAD, CW), jnp.float32),
    ],
    scratch_types=[
        pltpu.VMEM((BPW, LANES), jnp.int32),
        pltpu.VMEM((BPW, LANES), jnp.int32),
        pltpu.VMEM((GROUP, LANES, H), jnp.float32),
        pltpu.VMEM((LANES, CW), jnp.float32),
        pltpu.VMEM_SHARED((NPAD, H), jnp.float32),
        pltpu.VMEM_SHARED((NPAD, CW), jnp.float32),
        pltpu.SemaphoreType.DMA((GROUP,)),
        pltpu.SemaphoreType.DMA,
    ],
)
def _sc_agg_cnt(y_hbm, src_hbm, dst_hbm, zacc_hbm, zcnt_hbm, ones_hbm,
                out_acc, out_cnt, src_v, dst_v, bufs, ones_v, acc_s, cnt_s,
                sems, csem):
    c = lax.axis_index("c")
    s = lax.axis_index("s")
    base_b = jnp.where(c == 0, s * B0, NS * B0 + s * B1)
    nb_w = jnp.where(c == 0, B0, B1)
    r0 = s * RPT
    pltpu.sync_copy(zacc_hbm.at[pl.ds(r0, RPT)], acc_s.at[pl.ds(r0, RPT)])
    pltpu.sync_copy(zcnt_hbm.at[pl.ds(r0, RPT)], cnt_s.at[pl.ds(r0, RPT)])
    pltpu.sync_copy(ones_hbm, ones_v)
    pltpu.sync_copy(src_hbm.at[pl.ds(base_b, BPW)], src_v)
    pltpu.sync_copy(dst_hbm.at[pl.ds(base_b, BPW)], dst_v)
    plsc.subcore_barrier()

    ng = nb_w // GROUP
    for b in range(GROUP):
        pltpu.async_copy(y_hbm.at[src_v.at[b]], bufs.at[b], sems.at[b])

    def group(g, carry):
        base = g * GROUP
        for b in range(GROUP):
            pltpu.make_async_copy(y_hbm.at[src_v.at[base + b]], bufs.at[b],
                                  sems.at[b]).wait()
            pltpu.sync_copy(bufs.at[b], acc_s.at[dst_v.at[base + b]],
                            add=True)
            pltpu.async_copy(ones_v, cnt_s.at[dst_v.at[base + b]], csem,
                             add=True)

            @pl.when(g + 1 < ng)
            def _():
                pltpu.async_copy(y_hbm.at[src_v.at[base + GROUP + b]],
                                 bufs.at[b], sems.at[b])
        return carry

    lax.fori_loop(0, ng, group, 0)

    def drain(g, carry):
        base = g * GROUP
        for b in range(GROUP):
            pltpu.make_async_copy(ones_v, cnt_s.at[dst_v.at[base + b]],
                                  csem).wait()
        return carry

    lax.fori_loop(0, ng, drain, 0)
    plsc.subcore_barrier()
    pltpu.sync_copy(acc_s.at[pl.ds(r0, RPT)], out_acc.at[c, pl.ds(r0, RPT)])
    pltpu.sync_copy(cnt_s.at[pl.ds(r0, RPT)], out_cnt.at[c, pl.ds(r0, RPT)])


@functools.partial(
    pl.kernel,
    mesh=_sc_mesh(),
    compiler_params=pltpu.CompilerParams(use_tc_tiling_on_sc=False),
    out_type=jax.ShapeDtypeStruct((NC, NPAD, H), jnp.float32),
    scratch_types=[
        pltpu.VMEM((BPW, LANES), jnp.int32),
        pltpu.VMEM((BPW, LANES), jnp.int32),
        pltpu.VMEM((GROUP, LANES, H), jnp.float32),
        pltpu.VMEM_SHARED((NPAD, H), jnp.float32),
        pltpu.SemaphoreType.DMA((GROUP,)),
    ],
)
def _sc_agg(y_hbm, src_hbm, dst_hbm, zacc_hbm, out_acc, src_v, dst_v, bufs,
            acc_s, sems):
    c = lax.axis_index("c")
    s = lax.axis_index("s")
    base_b = jnp.where(c == 0, s * B0, NS * B0 + s * B1)
    nb_w = jnp.where(c == 0, B0, B1)
    r0 = s * RPT
    pltpu.sync_copy(zacc_hbm.at[pl.ds(r0, RPT)], acc_s.at[pl.ds(r0, RPT)])
    pltpu.sync_copy(src_hbm.at[pl.ds(base_b, BPW)], src_v)
    pltpu.sync_copy(dst_hbm.at[pl.ds(base_b, BPW)], dst_v)
    plsc.subcore_barrier()

    ng = nb_w // GROUP
    for b in range(GROUP):
        pltpu.async_copy(y_hbm.at[src_v.at[b]], bufs.at[b], sems.at[b])

    def group(g, carry):
        base = g * GROUP
        for b in range(GROUP):
            pltpu.make_async_copy(y_hbm.at[src_v.at[base + b]], bufs.at[b],
                                  sems.at[b]).wait()
            pltpu.sync_copy(bufs.at[b], acc_s.at[dst_v.at[base + b]],
                            add=True)

            @pl.when(g + 1 < ng)
            def _():
                pltpu.async_copy(y_hbm.at[src_v.at[base + GROUP + b]],
                                 bufs.at[b], sems.at[b])
        return carry

    lax.fori_loop(0, ng, group, 0)
    plsc.subcore_barrier()
    pltpu.sync_copy(acc_s.at[pl.ds(r0, RPT)], out_acc.at[c, pl.ds(r0, RPT)])


def _proj_body(x_ref, w_ref, oy_ref, or_ref):
    t = jnp.dot(x_ref[...], w_ref[...], preferred_element_type=jnp.float32)
    oy_ref[...] = t[:, :H]
    or_ref[...] = t[:, H:]


def _proj(x, wcat):
    return pl.pallas_call(
        _proj_body,
        grid=(N // ROWS_BLK,),
        in_specs=[
            pl.BlockSpec((ROWS_BLK, D), lambda i: (i, 0)),
            pl.BlockSpec((D, 2 * H), lambda i: (0, 0)),
        ],
        out_specs=[
            pl.BlockSpec((ROWS_BLK, H), lambda i: (i, 0)),
            pl.BlockSpec((ROWS_BLK, H), lambda i: (i, 0)),
        ],
        out_shape=[
            jax.ShapeDtypeStruct((N, H), jnp.float32),
            jax.ShapeDtypeStruct((N, H), jnp.float32),
        ],
    )(x, wcat)


def _mid_body(p_ref, c_ref, r_ref, b_ref, w_ref, oy_ref, or_ref, oc_ref):
    p = p_ref[...]
    cc = c_ref[...]
    cnt = jnp.maximum(cc[0] + cc[1], 1.0)
    agg = (p[0] + p[1]) / cnt[:, 0:1]
    h = jnp.maximum(agg + b_ref[...] + r_ref[...], 0.0)
    t = jnp.dot(h, w_ref[...], preferred_element_type=jnp.float32)
    oy_ref[...] = t[:, :H]
    or_ref[...] = t[:, H:]
    oc_ref[...] = cnt


def _mid(psum, pcnt, r1, b1, wcat2):
    blk_h = lambda i: (i, 0)
    return pl.pallas_call(
        _mid_body,
        grid=(N // ROWS_BLK,),
        in_specs=[
            pl.BlockSpec((NC, ROWS_BLK, H), lambda i: (0, i, 0)),
            pl.BlockSpec((NC, ROWS_BLK, CW), lambda i: (0, i, 0)),
            pl.BlockSpec((ROWS_BLK, H), blk_h),
            pl.BlockSpec((1, H), lambda i: (0, 0)),
            pl.BlockSpec((H, 2 * H), lambda i: (0, 0)),
        ],
        out_specs=[
            pl.BlockSpec((ROWS_BLK, H), blk_h),
            pl.BlockSpec((ROWS_BLK, H), blk_h),
            pl.BlockSpec((ROWS_BLK, CW), blk_h),
        ],
        out_shape=[
            jax.ShapeDtypeStruct((N, H), jnp.float32),
            jax.ShapeDtypeStruct((N, H), jnp.float32),
            jax.ShapeDtypeStruct((N, CW), jnp.float32),
        ],
    )(psum, pcnt, r1, b1, wcat2)


def _fin_body(p_ref, c_ref, r_ref, b_ref, o_ref):
    p = p_ref[...]
    o_ref[...] = ((p[0] + p[1]) / c_ref[:, 0:1] + b_ref[...] + r_ref[...])


def _fin(psum, cnt, r2, b2):
    blk_h = lambda i: (i, 0)
    return pl.pallas_call(
        _fin_body,
        grid=(N // ROWS_BLK,),
        in_specs=[
            pl.BlockSpec((NC, ROWS_BLK, H), lambda i: (0, i, 0)),
            pl.BlockSpec((ROWS_BLK, CW), blk_h),
            pl.BlockSpec((ROWS_BLK, H), blk_h),
            pl.BlockSpec((1, H), lambda i: (0, 0)),
        ],
        out_specs=pl.BlockSpec((ROWS_BLK, H), blk_h),
        out_shape=jax.ShapeDtypeStruct((N, H), jnp.float32),
    )(psum, cnt, r2, b2)


def kernel(x, edge_index, W1_l, b1, W1_r, W2_l, b2, W2_r):
    src = edge_index[0]
    dst = edge_index[1]
    src_p = jnp.concatenate([src, jnp.asarray(_PAD_SRC)]).reshape(NBP, LANES)
    dst_p = jnp.concatenate([dst, jnp.asarray(_PAD_DST)]).reshape(NBP, LANES)
    zacc = jnp.zeros((NPAD, H), jnp.float32)
    zcnt = jnp.zeros((NPAD, CW), jnp.float32)
    ones = jnp.ones((LANES, CW), jnp.float32)

    y1, r1 = _proj(x, jnp.concatenate([W1_l, W1_r], axis=1))
    psum1, pcnt = _sc_agg_cnt(y1, src_p, dst_p, zacc, zcnt, ones)
    y2, r2, cnt = _mid(psum1, pcnt, r1, b1.reshape(1, H),
                       jnp.concatenate([W2_l, W2_r], axis=1))

    psum2 = _sc_agg(y2, src_p, dst_p, zacc)
    return _fin(psum2, cnt, r2, b2.reshape(1, H))


# speedup vs baseline: 1.3882x; 1.3272x over previous
import functools

import numpy as np

import jax
import jax.numpy as jnp
from jax import lax
from jax.experimental import pallas as pl
from jax.experimental.pallas import tpu as pltpu
from jax.experimental.pallas import tpu_sc as plsc

N = 10000
D = 128
H = 32
E = 320000

LANES = 128
GROUP = 16
NC = 2
NS = 16
NW = NC * NS
B0 = 80
B1 = 80
BPW = max(B0, B1)
NB = NS * (B0 + B1)
NBP = NB + (BPW - min(B0, B1))
EP = NBP * LANES
NPAD = 10112
RPT = NPAD // NS
DUMMY = 10104
CW = 16

ROWS_BLK = 2000

_PAD_SRC = np.zeros((EP - E,), np.int32)
_PAD_DST = (N + np.arange(EP - E, dtype=np.int32) % (NPAD - N)).astype(np.int32)


def _sc_mesh():
    return plsc.VectorSubcoreMesh(core_axis_name="c", subcore_axis_name="s")


@functools.partial(
    pl.kernel,
    mesh=_sc_mesh(),
    compiler_params=pltpu.CompilerParams(use_tc_tiling_on_sc=False),
    out_type=[
        jax.ShapeDtypeStruct((NC, NPAD, H), jnp.float32),
        jax.ShapeDtypeStruct((NC, NPAD, CW), jnp.float32),
    ],
    scratch_types=[
        pltpu.VMEM((BPW, LANES), jnp.int32),
        pltpu.VMEM((BPW, LANES), jnp.int32),
        pltpu.VMEM((GROUP, LANES, H), jnp.float32),
        pltpu.VMEM((LANES, CW), jnp.float32),
        pltpu.VMEM_SHARED((NPAD, H), jnp.float32),
        pltpu.VMEM_SHARED((NPAD, CW), jnp.float32),
        pltpu.SemaphoreType.DMA((GROUP,)),
        pltpu.SemaphoreType.DMA,
    ],
)
def _sc_agg_cnt(y_hbm, src_hbm, dst_hbm, zacc_hbm, zcnt_hbm, ones_hbm,
                out_acc, out_cnt, src_v, dst_v, bufs, ones_v, acc_s, cnt_s,
                sems, csem):
    c = lax.axis_index("c")
    s = lax.axis_index("s")
    base_b = jnp.where(c == 0, s * B0, NS * B0 + s * B1)
    nb_w = jnp.where(c == 0, B0, B1)
    r0 = s * RPT
    pltpu.sync_copy(zacc_hbm.at[pl.ds(r0, RPT)], acc_s.at[pl.ds(r0, RPT)])
    pltpu.sync_copy(zcnt_hbm.at[pl.ds(r0, RPT)], cnt_s.at[pl.ds(r0, RPT)])
    pltpu.sync_copy(ones_hbm, ones_v)
    pltpu.sync_copy(src_hbm.at[pl.ds(base_b, BPW)], src_v)
    pltpu.sync_copy(dst_hbm.at[pl.ds(base_b, BPW)], dst_v)
    plsc.subcore_barrier()

    ng = nb_w // GROUP
    for b in range(GROUP):
        pltpu.async_copy(y_hbm.at[src_v.at[b]], bufs.at[b], sems.at[b])

    def group(g, carry):
        base = g * GROUP
        for b in range(GROUP):
            pltpu.make_async_copy(y_hbm.at[src_v.at[base + b]], bufs.at[b],
                                  sems.at[b]).wait()
            pltpu.sync_copy(bufs.at[b], acc_s.at[dst_v.at[base + b]],
                            add=True)
            pltpu.async_copy(ones_v, cnt_s.at[dst_v.at[base + b]], csem,
                             add=True)

            @pl.when(g + 1 < ng)
            def _():
                pltpu.async_copy(y_hbm.at[src_v.at[base + GROUP + b]],
                                 bufs.at[b], sems.at[b])
        return carry

    lax.fori_loop(0, ng, group, 0)

    def drain(g, carry):
        base = g * GROUP
        for b in range(GROUP):
            pltpu.make_async_copy(ones_v, cnt_s.at[dst_v.at[base + b]],
                                  csem).wait()
        return carry

    lax.fori_loop(0, ng, drain, 0)
    plsc.subcore_barrier()
    pltpu.sync_copy(acc_s.at[pl.ds(r0, RPT)], out_acc.at[c, pl.ds(r0, RPT)])
    pltpu.sync_copy(cnt_s.at[pl.ds(r0, RPT)], out_cnt.at[c, pl.ds(r0, RPT)])


@functools.partial(
    pl.kernel,
    mesh=_sc_mesh(),
    compiler_params=pltpu.CompilerParams(use_tc_tiling_on_sc=False),
    out_type=jax.ShapeDtypeStruct((NC, NPAD, H), jnp.float32),
    scratch_types=[
        pltpu.VMEM((BPW, LANES), jnp.int32),
        pltpu.VMEM((BPW, LANES), jnp.int32),
        pltpu.VMEM((GROUP, LANES, H), jnp.float32),
        pltpu.VMEM_SHARED((NPAD, H), jnp.float32),
        pltpu.VMEM_SHARED((NPAD, H), jnp.float32),
        pltpu.SemaphoreType.DMA((GROUP,)),
    ],
)
def _sc_agg(y_hbm, src_hbm, dst_hbm, zacc_hbm, out_acc, src_v, dst_v, bufs,
            acc_s, y_s, sems):
    c = lax.axis_index("c")
    s = lax.axis_index("s")
    base_b = jnp.where(c == 0, s * B0, NS * B0 + s * B1)
    nb_w = jnp.where(c == 0, B0, B1)
    r0 = s * RPT
    pltpu.sync_copy(zacc_hbm.at[pl.ds(r0, RPT)], acc_s.at[pl.ds(r0, RPT)])
    pltpu.sync_copy(y_hbm.at[pl.ds(r0, RPT)], y_s.at[pl.ds(r0, RPT)])
    pltpu.sync_copy(src_hbm.at[pl.ds(base_b, BPW)], src_v)
    pltpu.sync_copy(dst_hbm.at[pl.ds(base_b, BPW)], dst_v)
    plsc.subcore_barrier()

    ng = nb_w // GROUP
    for b in range(GROUP):
        pltpu.async_copy(y_s.at[src_v.at[b]], bufs.at[b], sems.at[b])

    def group(g, carry):
        base = g * GROUP
        for b in range(GROUP):
            pltpu.make_async_copy(y_s.at[src_v.at[base + b]], bufs.at[b],
                                  sems.at[b]).wait()
            pltpu.sync_copy(bufs.at[b], acc_s.at[dst_v.at[base + b]],
                            add=True)

            @pl.when(g + 1 < ng)
            def _():
                pltpu.async_copy(y_s.at[src_v.at[base + GROUP + b]],
                                 bufs.at[b], sems.at[b])
        return carry

    lax.fori_loop(0, ng, group, 0)
    plsc.subcore_barrier()
    pltpu.sync_copy(acc_s.at[pl.ds(r0, RPT)], out_acc.at[c, pl.ds(r0, RPT)])


def _proj_body(x_ref, w_ref, oy_ref, or_ref):
    t = jnp.dot(x_ref[...], w_ref[...], preferred_element_type=jnp.float32)
    oy_ref[...] = t[:, :H]
    or_ref[...] = t[:, H:]


def _proj(x, wcat):
    return pl.pallas_call(
        _proj_body,
        grid=(N // ROWS_BLK,),
        in_specs=[
            pl.BlockSpec((ROWS_BLK, D), lambda i: (i, 0)),
            pl.BlockSpec((D, 2 * H), lambda i: (0, 0)),
        ],
        out_specs=[
            pl.BlockSpec((ROWS_BLK, H), lambda i: (i, 0)),
            pl.BlockSpec((ROWS_BLK, H), lambda i: (i, 0)),
        ],
        out_shape=[
            jax.ShapeDtypeStruct((NPAD, H), jnp.float32),
            jax.ShapeDtypeStruct((N, H), jnp.float32),
        ],
    )(x, wcat)


def _mid_body(p_ref, c_ref, r_ref, b_ref, w_ref, oy_ref, or_ref, oc_ref):
    p = p_ref[...]
    cc = c_ref[...]
    cnt = jnp.maximum(cc[0] + cc[1], 1.0)
    agg = (p[0] + p[1]) / cnt[:, 0:1]
    h = jnp.maximum(agg + b_ref[...] + r_ref[...], 0.0)
    t = jnp.dot(h, w_ref[...], preferred_element_type=jnp.float32)
    oy_ref[...] = t[:, :H]
    or_ref[...] = t[:, H:]
    oc_ref[...] = cnt


def _mid(psum, pcnt, r1, b1, wcat2):
    blk_h = lambda i: (i, 0)
    return pl.pallas_call(
        _mid_body,
        grid=(N // ROWS_BLK,),
        in_specs=[
            pl.BlockSpec((NC, ROWS_BLK, H), lambda i: (0, i, 0)),
            pl.BlockSpec((NC, ROWS_BLK, CW), lambda i: (0, i, 0)),
            pl.BlockSpec((ROWS_BLK, H), blk_h),
            pl.BlockSpec((1, H), lambda i: (0, 0)),
            pl.BlockSpec((H, 2 * H), lambda i: (0, 0)),
        ],
        out_specs=[
            pl.BlockSpec((ROWS_BLK, H), blk_h),
            pl.BlockSpec((ROWS_BLK, H), blk_h),
            pl.BlockSpec((ROWS_BLK, CW), blk_h),
        ],
        out_shape=[
            jax.ShapeDtypeStruct((NPAD, H), jnp.float32),
            jax.ShapeDtypeStruct((N, H), jnp.float32),
            jax.ShapeDtypeStruct((N, CW), jnp.float32),
        ],
    )(psum, pcnt, r1, b1, wcat2)


def _fin_body(p_ref, c_ref, r_ref, b_ref, o_ref):
    p = p_ref[...]
    o_ref[...] = ((p[0] + p[1]) / c_ref[:, 0:1] + b_ref[...] + r_ref[...])


def _fin(psum, cnt, r2, b2):
    blk_h = lambda i: (i, 0)
    return pl.pallas_call(
        _fin_body,
        grid=(N // ROWS_BLK,),
        in_specs=[
            pl.BlockSpec((NC, ROWS_BLK, H), lambda i: (0, i, 0)),
            pl.BlockSpec((ROWS_BLK, CW), blk_h),
            pl.BlockSpec((ROWS_BLK, H), blk_h),
            pl.BlockSpec((1, H), lambda i: (0, 0)),
        ],
        out_specs=pl.BlockSpec((ROWS_BLK, H), blk_h),
        out_shape=jax.ShapeDtypeStruct((N, H), jnp.float32),
    )(psum, cnt, r2, b2)


def kernel(x, edge_index, W1_l, b1, W1_r, W2_l, b2, W2_r):
    src = edge_index[0]
    dst = edge_index[1]
    src_p = jnp.concatenate([src, jnp.asarray(_PAD_SRC)]).reshape(NBP, LANES)
    dst_p = jnp.concatenate([dst, jnp.asarray(_PAD_DST)]).reshape(NBP, LANES)
    zacc = jnp.zeros((NPAD, H), jnp.float32)
    zcnt = jnp.zeros((NPAD, CW), jnp.float32)
    ones = jnp.ones((LANES, CW), jnp.float32)

    y1, r1 = _proj(x, jnp.concatenate([W1_l, W1_r], axis=1))
    psum1, pcnt = _sc_agg_cnt(y1, src_p, dst_p, zacc, zcnt, ones)
    y2, r2, cnt = _mid(psum1, pcnt, r1, b1.reshape(1, H),
                       jnp.concatenate([W2_l, W2_r], axis=1))

    psum2 = _sc_agg(y2, src_p, dst_p, zacc)
    return _fin(psum2, cnt, r2, b2.reshape(1, H))


# speedup vs baseline: 2.0521x; 1.4783x over previous
import functools

import numpy as np

import jax
import jax.numpy as jnp
from jax import lax
from jax.experimental import pallas as pl
from jax.experimental.pallas import tpu as pltpu
from jax.experimental.pallas import tpu_sc as plsc

N = 10000
D = 128
H = 32
E = 320000

LANES = 128
GROUP = 16
NC = 2
NS = 16
NW = NC * NS
B0 = 80
B1 = 80
BPW = max(B0, B1)
NB = NS * (B0 + B1)
NBP = NB + (BPW - min(B0, B1))
EP = NBP * LANES
NPAD = 10112
RPT = NPAD // NS
DUMMY = 10104
CW = 16

ROWS_BLK = 2000

_PAD_SRC = np.zeros((EP - E,), np.int32)
_PAD_DST = (N + np.arange(EP - E, dtype=np.int32) % (NPAD - N)).astype(np.int32)


def _sc_mesh():
    return plsc.VectorSubcoreMesh(core_axis_name="c", subcore_axis_name="s")


@functools.partial(
    pl.kernel,
    mesh=_sc_mesh(),
    compiler_params=pltpu.CompilerParams(use_tc_tiling_on_sc=False),
    out_type=jax.ShapeDtypeStruct((NC, NPAD, CW), jnp.float32),
    scratch_types=[
        pltpu.VMEM((BPW, LANES), jnp.int32),
        pltpu.VMEM((LANES, CW), jnp.float32),
        pltpu.VMEM_SHARED((NPAD, CW), jnp.float32),
        pltpu.SemaphoreType.DMA,
    ],
)
def _sc_cnt(dst_hbm, zcnt_hbm, ones_hbm, out_cnt, dst_v, ones_v, cnt_s,
            csem):
    c = lax.axis_index("c")
    s = lax.axis_index("s")
    base_b = jnp.where(c == 0, s * B0, NS * B0 + s * B1)
    nb_w = jnp.where(c == 0, B0, B1)
    r0 = s * RPT
    pltpu.sync_copy(zcnt_hbm.at[pl.ds(r0, RPT)], cnt_s.at[pl.ds(r0, RPT)])
    pltpu.sync_copy(ones_hbm, ones_v)
    pltpu.sync_copy(dst_hbm.at[pl.ds(base_b, BPW)], dst_v)
    plsc.subcore_barrier()

    ng = nb_w // GROUP

    def group(g, carry):
        base = g * GROUP
        for b in range(GROUP):
            pltpu.async_copy(ones_v, cnt_s.at[dst_v.at[base + b]], csem,
                             add=True)
        return carry

    lax.fori_loop(0, ng, group, 0)

    def drain(g, carry):
        base = g * GROUP
        for b in range(GROUP):
            pltpu.make_async_copy(ones_v, cnt_s.at[dst_v.at[base + b]],
                                  csem).wait()
        return carry

    lax.fori_loop(0, ng, drain, 0)
    plsc.subcore_barrier()
    pltpu.sync_copy(cnt_s.at[pl.ds(r0, RPT)], out_cnt.at[c, pl.ds(r0, RPT)])


@functools.partial(
    pl.kernel,
    mesh=_sc_mesh(),
    compiler_params=pltpu.CompilerParams(use_tc_tiling_on_sc=False),
    out_type=jax.ShapeDtypeStruct((NC, NPAD, H), jnp.float32),
    scratch_types=[
        pltpu.VMEM((BPW, LANES), jnp.int32),
        pltpu.VMEM((BPW, LANES), jnp.int32),
        pltpu.VMEM((GROUP, LANES, H), jnp.float32),
        pltpu.VMEM_SHARED((NPAD, H), jnp.float32),
        pltpu.VMEM_SHARED((NPAD, H), jnp.float32),
        pltpu.SemaphoreType.DMA((GROUP,)),
    ],
)
def _sc_agg(y_hbm, src_hbm, dst_hbm, zacc_hbm, out_acc, src_v, dst_v, bufs,
            acc_s, y_s, sems):
    c = lax.axis_index("c")
    s = lax.axis_index("s")
    base_b = jnp.where(c == 0, s * B0, NS * B0 + s * B1)
    nb_w = jnp.where(c == 0, B0, B1)
    r0 = s * RPT
    pltpu.sync_copy(zacc_hbm.at[pl.ds(r0, RPT)], acc_s.at[pl.ds(r0, RPT)])
    pltpu.sync_copy(y_hbm.at[pl.ds(r0, RPT)], y_s.at[pl.ds(r0, RPT)])
    pltpu.sync_copy(src_hbm.at[pl.ds(base_b, BPW)], src_v)
    pltpu.sync_copy(dst_hbm.at[pl.ds(base_b, BPW)], dst_v)
    plsc.subcore_barrier()

    ng = nb_w // GROUP
    for b in range(GROUP):
        pltpu.async_copy(y_s.at[src_v.at[b]], bufs.at[b], sems.at[b])

    def group(g, carry):
        base = g * GROUP
        for b in range(GROUP):
            pltpu.make_async_copy(y_s.at[src_v.at[base + b]], bufs.at[b],
                                  sems.at[b]).wait()
            pltpu.sync_copy(bufs.at[b], acc_s.at[dst_v.at[base + b]],
                            add=True)

            @pl.when(g + 1 < ng)
            def _():
                pltpu.async_copy(y_s.at[src_v.at[base + GROUP + b]],
                                 bufs.at[b], sems.at[b])
        return carry

    lax.fori_loop(0, ng, group, 0)
    plsc.subcore_barrier()
    pltpu.sync_copy(acc_s.at[pl.ds(r0, RPT)], out_acc.at[c, pl.ds(r0, RPT)])


def _proj_body(x_ref, w_ref, oy_ref, or_ref):
    t = jnp.dot(x_ref[...], w_ref[...], preferred_element_type=jnp.float32)
    oy_ref[...] = t[:, :H]
    or_ref[...] = t[:, H:]


def _proj(x, wcat):
    return pl.pallas_call(
        _proj_body,
        grid=(N // ROWS_BLK,),
        in_specs=[
            pl.BlockSpec((ROWS_BLK, D), lambda i: (i, 0)),
            pl.BlockSpec((D, 2 * H), lambda i: (0, 0)),
        ],
        out_specs=[
            pl.BlockSpec((ROWS_BLK, H), lambda i: (i, 0)),
            pl.BlockSpec((ROWS_BLK, H), lambda i: (i, 0)),
        ],
        out_shape=[
            jax.ShapeDtypeStruct((NPAD, H), jnp.float32),
            jax.ShapeDtypeStruct((N, H), jnp.float32),
        ],
    )(x, wcat)


def _mid_body(p_ref, c_ref, r_ref, b_ref, w_ref, oy_ref, or_ref, oc_ref):
    p = p_ref[...]
    cc = c_ref[...]
    cnt = jnp.maximum(cc[0] + cc[1], 1.0)
    agg = (p[0] + p[1]) / cnt[:, 0:1]
    h = jnp.maximum(agg + b_ref[...] + r_ref[...], 0.0)
    t = jnp.dot(h, w_ref[...], preferred_element_type=jnp.float32)
    oy_ref[...] = t[:, :H]
    or_ref[...] = t[:, H:]
    oc_ref[...] = cnt


def _mid(psum, pcnt, r1, b1, wcat2):
    blk_h = lambda i: (i, 0)
    return pl.pallas_call(
        _mid_body,
        grid=(N // ROWS_BLK,),
        in_specs=[
            pl.BlockSpec((NC, ROWS_BLK, H), lambda i: (0, i, 0)),
            pl.BlockSpec((NC, ROWS_BLK, CW), lambda i: (0, i, 0)),
            pl.BlockSpec((ROWS_BLK, H), blk_h),
            pl.BlockSpec((1, H), lambda i: (0, 0)),
            pl.BlockSpec((H, 2 * H), lambda i: (0, 0)),
        ],
        out_specs=[
            pl.BlockSpec((ROWS_BLK, H), blk_h),
            pl.BlockSpec((ROWS_BLK, H), blk_h),
            pl.BlockSpec((ROWS_BLK, CW), blk_h),
        ],
        out_shape=[
            jax.ShapeDtypeStruct((NPAD, H), jnp.float32),
            jax.ShapeDtypeStruct((N, H), jnp.float32),
            jax.ShapeDtypeStruct((N, CW), jnp.float32),
        ],
    )(psum, pcnt, r1, b1, wcat2)


def _fin_body(p_ref, c_ref, r_ref, b_ref, o_ref):
    p = p_ref[...]
    o_ref[...] = ((p[0] + p[1]) / c_ref[:, 0:1] + b_ref[...] + r_ref[...])


def _fin(psum, cnt, r2, b2):
    blk_h = lambda i: (i, 0)
    return pl.pallas_call(
        _fin_body,
        grid=(N // ROWS_BLK,),
        in_specs=[
            pl.BlockSpec((NC, ROWS_BLK, H), lambda i: (0, i, 0)),
            pl.BlockSpec((ROWS_BLK, CW), blk_h),
            pl.BlockSpec((ROWS_BLK, H), blk_h),
            pl.BlockSpec((1, H), lambda i: (0, 0)),
        ],
        out_specs=pl.BlockSpec((ROWS_BLK, H), blk_h),
        out_shape=jax.ShapeDtypeStruct((N, H), jnp.float32),
    )(psum, cnt, r2, b2)


def kernel(x, edge_index, W1_l, b1, W1_r, W2_l, b2, W2_r):
    src = edge_index[0]
    dst = edge_index[1]
    src_p = jnp.concatenate([src, jnp.asarray(_PAD_SRC)]).reshape(NBP, LANES)
    dst_p = jnp.concatenate([dst, jnp.asarray(_PAD_DST)]).reshape(NBP, LANES)
    zacc = jnp.zeros((NPAD, H), jnp.float32)
    zcnt = jnp.zeros((NPAD, CW), jnp.float32)
    ones = jnp.ones((LANES, CW), jnp.float32)

    pcnt = _sc_cnt(dst_p, zcnt, ones)
    y1, r1 = _proj(x, jnp.concatenate([W1_l, W1_r], axis=1))
    psum1 = _sc_agg(y1, src_p, dst_p, zacc)
    y2, r2, cnt = _mid(psum1, pcnt, r1, b1.reshape(1, H),
                       jnp.concatenate([W2_l, W2_r], axis=1))

    psum2 = _sc_agg(y2, src_p, dst_p, zacc)
    return _fin(psum2, cnt, r2, b2.reshape(1, H))
